# baseline (speedup 1.0000x reference)
import functools

import jax
import jax.numpy as jnp
from jax import lax
from jax.experimental import pallas as pl
from jax.experimental.pallas import tpu as pltpu
from jax.experimental.pallas import tpu_sc as plsc


def _sc_pane_gather(xcatT, Et, B, F, D, V):
    info = plsc.get_sparse_core_info()
    NC, NS = info.num_cores, info.num_subcores
    NW = NC * NS
    P = F * D
    per_w = P // NW
    CH = 4096
    NCH = B // CH

    mesh = plsc.VectorSubcoreMesh(core_axis_name="c", subcore_axis_name="s")

    @functools.partial(
        pl.kernel,
        mesh=mesh,
        compiler_params=pltpu.CompilerParams(needs_layout_passes=False),
        out_type=jax.ShapeDtypeStruct((P, B), jnp.float32),
        scratch_types=[
            pltpu.VMEM((V,), jnp.float32),
            pltpu.VMEM((B,), jnp.int32),
            pltpu.VMEM((2, CH), jnp.float32),
            pltpu.SemaphoreType.DMA,
            pltpu.SemaphoreType.DMA,
        ],
    )
    def pane_kernel(xcatT_hbm, et_hbm, out_hbm, row_v, idx_v, out_v, rsem, wsem):
        wid = lax.axis_index("s") * NC + lax.axis_index("c")

        def pair_loop(pi, carry):
            p = wid * per_w + pi
            f = p // D
            d = p % D
            cp = pltpu.async_copy(et_hbm.at[f, d], row_v, rsem)
            pltpu.sync_copy(xcatT_hbm.at[f], idx_v)
            cp.wait()

            def chunk_loop(h, carry2):
                b = h % 2

                @pl.when(pi * NCH + h >= 2)
                def _():
                    pltpu.make_async_copy(
                        out_v.at[0], out_hbm.at[0, pl.ds(0, CH)], wsem).wait()

                def g(i, carry3):
                    for k in range(8):
                        o = (i * 8 + k) * 16
                        idx16 = idx_v[pl.ds(h * CH + o, 16)]
                        out_v[b, pl.ds(o, 16)] = plsc.load_gather(row_v, [idx16])
                    return carry3

                lax.fori_loop(0, CH // 128, g, 0)
                pltpu.async_copy(
                    out_v.at[b], out_hbm.at[p, pl.ds(h * CH, CH)], wsem)
                return carry2

            lax.fori_loop(0, NCH, chunk_loop, 0)
            return carry

        lax.fori_loop(0, per_w, pair_loop, 0)
        for _ in range(2):
            pltpu.make_async_copy(
                out_v.at[0], out_hbm.at[0, pl.ds(0, CH)], wsem).wait()

    return pane_kernel(xcatT, Et)


def _mlp_body(xn_ref, emb_ref, w1n_ref, w1e_ref, b1_ref, w2_ref, b2_ref,
              w3_ref, b3_ref, wab_ref, bab_ref, out_ref):
    h = jnp.dot(w1e_ref[...], emb_ref[...], preferred_element_type=jnp.float32)
    h = h + jnp.dot(w1n_ref[...], xn_ref[...], preferred_element_type=jnp.float32)
    h = jnp.maximum(h + b1_ref[...], 0.0)
    h = jnp.maximum(
        jnp.dot(w2_ref[...], h, preferred_element_type=jnp.float32) + b2_ref[...], 0.0)
    h = jnp.maximum(
        jnp.dot(w3_ref[...], h, preferred_element_type=jnp.float32) + b3_ref[...], 0.0)
    out_ref[...] = jnp.dot(wab_ref[...], h, preferred_element_type=jnp.float32) + bab_ref[...]


def _mlp_t(xnT, embT, w1nT, w1eT, b1, w2T, b2, w3T, b3, wabT, bab):
    ED, B = embT.shape
    ND = xnT.shape[0]
    H1, H2, H3 = w2T.shape[1], w3T.shape[1], wabT.shape[1]
    BM = 2048
    grid = (B // BM,)
    full = lambda shape: pl.BlockSpec(shape, lambda i: (0, 0))
    return pl.pallas_call(
        _mlp_body,
        grid=grid,
        in_specs=[
            pl.BlockSpec((ND, BM), lambda i: (0, i)),
            pl.BlockSpec((ED, BM), lambda i: (0, i)),
            full((H1, ND)),
            full((H1, ED)),
            full((H1, 1)),
            full((H2, H1)),
            full((H2, 1)),
            full((H3, H2)),
            full((H3, 1)),
            full((2, H3)),
            full((2, 1)),
        ],
        out_specs=pl.BlockSpec((2, BM), lambda i: (0, i)),
        out_shape=jax.ShapeDtypeStruct((2, B), jnp.float32),
    )(xnT, embT, w1nT, w1eT, b1, w2T, b2, w3T, b3, wabT, bab)


def kernel(x_num, x_cat, E, W1, b1, W2, b2, W3, b3, WA, bA, WB, bB):
    B, F = x_cat.shape
    _, V, D = E.shape
    Et = jnp.transpose(E, (0, 2, 1))
    xcatT = x_cat.T
    embT = _sc_pane_gather(xcatT, Et, B, F, D, V)

    nd = x_num.shape[1]
    w1n, w1e = W1[:nd], W1[nd:]
    wab = jnp.concatenate([WA, WB], axis=1)
    bab = jnp.concatenate([bA, bB])[:, None]
    out = _mlp_t(x_num.T, embT, w1n.T, w1e.T, b1[:, None], W2.T, b2[:, None],
                 W3.T, b3[:, None], wab.T, bab)
    return out[0], out[1]


# speedup vs baseline: 1.2288x; 1.2288x over previous
import functools

import jax
import jax.numpy as jnp
from jax import lax
from jax.experimental import pallas as pl
from jax.experimental.pallas import tpu as pltpu
from jax.experimental.pallas import tpu_sc as plsc


def _sc_pane_gather(xcatT, Et, B, F, D, V):
    info = plsc.get_sparse_core_info()
    NC, NS = info.num_cores, info.num_subcores
    NW = NC * NS
    P = F * D
    per_w = P // NW
    CH = 8192
    NCH = B // CH

    mesh = plsc.VectorSubcoreMesh(core_axis_name="c", subcore_axis_name="s")

    @functools.partial(
        pl.kernel,
        mesh=mesh,
        compiler_params=pltpu.CompilerParams(needs_layout_passes=False),
        out_type=jax.ShapeDtypeStruct((P, B), jnp.float32),
        scratch_types=[
            pltpu.VMEM((V,), jnp.float32),
            pltpu.VMEM((CH,), jnp.int32),
            pltpu.VMEM((CH,), jnp.float32),
        ],
    )
    def pane_kernel(xcatT_hbm, et_hbm, out_hbm, row_v, idx_v, out_v):
        wid = lax.axis_index("s") * NC + lax.axis_index("c")

        def pair_loop(pi, carry):
            p = wid * per_w + pi
            f = p // D
            d = p % D
            pltpu.sync_copy(et_hbm.at[f, d], row_v)

            def chunk_loop(h, carry2):
                pltpu.sync_copy(xcatT_hbm.at[f, pl.ds(h * CH, CH)], idx_v)

                def g(i, carry3):
                    for k in range(4):
                        o = (i * 4 + k) * 16
                        idx16 = idx_v[pl.ds(o, 16)]
                        out_v[pl.ds(o, 16)] = plsc.load_gather(row_v, [idx16])
                    return carry3

                lax.fori_loop(0, CH // 64, g, 0)
                pltpu.sync_copy(out_v, out_hbm.at[p, pl.ds(h * CH, CH)])
                return carry2

            lax.fori_loop(0, NCH, chunk_loop, 0)
            return carry

        lax.fori_loop(0, per_w, pair_loop, 0)

    return pane_kernel(xcatT, Et)


def _mlp_body(xn_ref, emb_ref, w1n_ref, w1e_ref, b1_ref, w2_ref, b2_ref,
              w3_ref, b3_ref, wab_ref, bab_ref, out_ref):
    h = jnp.dot(w1e_ref[...], emb_ref[...], preferred_element_type=jnp.float32)
    h = h + jnp.dot(w1n_ref[...], xn_ref[...], preferred_element_type=jnp.float32)
    h = jnp.maximum(h + b1_ref[...], 0.0)
    h = jnp.maximum(
        jnp.dot(w2_ref[...], h, preferred_element_type=jnp.float32) + b2_ref[...], 0.0)
    h = jnp.maximum(
        jnp.dot(w3_ref[...], h, preferred_element_type=jnp.float32) + b3_ref[...], 0.0)
    out_ref[...] = jnp.dot(wab_ref[...], h, preferred_element_type=jnp.float32) + bab_ref[...]


def _mlp_t(xnT, embT, w1nT, w1eT, b1, w2T, b2, w3T, b3, wabT, bab):
    ED, B = embT.shape
    ND = xnT.shape[0]
    H1, H2, H3 = w2T.shape[1], w3T.shape[1], wabT.shape[1]
    BM = 2048
    grid = (B // BM,)
    full = lambda shape: pl.BlockSpec(shape, lambda i: (0, 0))
    return pl.pallas_call(
        _mlp_body,
        grid=grid,
        in_specs=[
            pl.BlockSpec((ND, BM), lambda i: (0, i)),
            pl.BlockSpec((ED, BM), lambda i: (0, i)),
            full((H1, ND)),
            full((H1, ED)),
            full((H1, 1)),
            full((H2, H1)),
            full((H2, 1)),
            full((H3, H2)),
            full((H3, 1)),
            full((2, H3)),
            full((2, 1)),
        ],
        out_specs=pl.BlockSpec((2, BM), lambda i: (0, i)),
        out_shape=jax.ShapeDtypeStruct((2, B), jnp.float32),
    )(xnT, embT, w1nT, w1eT, b1, w2T, b2, w3T, b3, wabT, bab)


def kernel(x_num, x_cat, E, W1, b1, W2, b2, W3, b3, WA, bA, WB, bB):
    B, F = x_cat.shape
    _, V, D = E.shape
    Et = jnp.transpose(E, (0, 2, 1))
    xcatT = x_cat.T
    embT = _sc_pane_gather(xcatT, Et, B, F, D, V)

    nd = x_num.shape[1]
    w1n, w1e = W1[:nd], W1[nd:]
    wab = jnp.concatenate([WA, WB], axis=1)
    bab = jnp.concatenate([bA, bB])[:, None]
    out = _mlp_t(x_num.T, embT, w1n.T, w1e.T, b1[:, None], W2.T, b2[:, None],
                 W3.T, b3[:, None], wab.T, bab)
    return out[0], out[1]


# speedup vs baseline: 1.5279x; 1.2434x over previous
import functools

import jax
import jax.numpy as jnp
from jax import lax
from jax.experimental import pallas as pl
from jax.experimental.pallas import tpu as pltpu
from jax.experimental.pallas import tpu_sc as plsc


def _sc_pane_gather(xcatT, Et, B, F, D, V):
    info = plsc.get_sparse_core_info()
    NC, NS = info.num_cores, info.num_subcores
    NW = NC * NS
    P = F * D
    per_w = P // NW
    CH = 8192
    NCH = B // CH

    mesh = plsc.VectorSubcoreMesh(core_axis_name="c", subcore_axis_name="s")

    @functools.partial(
        pl.kernel,
        mesh=mesh,
        compiler_params=pltpu.CompilerParams(needs_layout_passes=False),
        out_type=jax.ShapeDtypeStruct((P, B), jnp.float32),
        scratch_types=[
            pltpu.VMEM((V,), jnp.float32),
            pltpu.VMEM((CH,), jnp.int32),
            pltpu.VMEM((CH,), jnp.float32),
        ],
    )
    def pane_kernel(xcatT_hbm, et_hbm, out_hbm, row_v, idx_v, out_v):
        wid = lax.axis_index("s") * NC + lax.axis_index("c")

        def pair_loop(pi, carry):
            p = wid * per_w + pi
            f = p // D
            d = p % D
            pltpu.sync_copy(et_hbm.at[f, d], row_v)

            def chunk_loop(h, carry2):
                pltpu.sync_copy(xcatT_hbm.at[f, pl.ds(h * CH, CH)], idx_v)

                @plsc.parallel_loop(0, CH, step=16, unroll=8)
                def _(o):
                    idx16 = idx_v[pl.ds(o, 16)]
                    out_v[pl.ds(o, 16)] = plsc.load_gather(row_v, [idx16])
                pltpu.sync_copy(out_v, out_hbm.at[p, pl.ds(h * CH, CH)])
                return carry2

            lax.fori_loop(0, NCH, chunk_loop, 0)
            return carry

        lax.fori_loop(0, per_w, pair_loop, 0)

    return pane_kernel(xcatT, Et)


def _mlp_body(xn_ref, emb_ref, w1n_ref, w1e_ref, b1_ref, w2_ref, b2_ref,
              w3_ref, b3_ref, wab_ref, bab_ref, out_ref):
    h = jnp.dot(w1e_ref[...], emb_ref[...], preferred_element_type=jnp.float32)
    h = h + jnp.dot(w1n_ref[...], xn_ref[...], preferred_element_type=jnp.float32)
    h = jnp.maximum(h + b1_ref[...], 0.0)
    h = jnp.maximum(
        jnp.dot(w2_ref[...], h, preferred_element_type=jnp.float32) + b2_ref[...], 0.0)
    h = jnp.maximum(
        jnp.dot(w3_ref[...], h, preferred_element_type=jnp.float32) + b3_ref[...], 0.0)
    out_ref[...] = jnp.dot(wab_ref[...], h, preferred_element_type=jnp.float32) + bab_ref[...]


def _mlp_t(xnT, embT, w1nT, w1eT, b1, w2T, b2, w3T, b3, wabT, bab):
    ED, B = embT.shape
    ND = xnT.shape[0]
    H1, H2, H3 = w2T.shape[1], w3T.shape[1], wabT.shape[1]
    BM = 2048
    grid = (B // BM,)
    full = lambda shape: pl.BlockSpec(shape, lambda i: (0, 0))
    return pl.pallas_call(
        _mlp_body,
        grid=grid,
        in_specs=[
            pl.BlockSpec((ND, BM), lambda i: (0, i)),
            pl.BlockSpec((ED, BM), lambda i: (0, i)),
            full((H1, ND)),
            full((H1, ED)),
            full((H1, 1)),
            full((H2, H1)),
            full((H2, 1)),
            full((H3, H2)),
            full((H3, 1)),
            full((2, H3)),
            full((2, 1)),
        ],
        out_specs=pl.BlockSpec((2, BM), lambda i: (0, i)),
        out_shape=jax.ShapeDtypeStruct((2, B), jnp.float32),
    )(xnT, embT, w1nT, w1eT, b1, w2T, b2, w3T, b3, wabT, bab)


def kernel(x_num, x_cat, E, W1, b1, W2, b2, W3, b3, WA, bA, WB, bB):
    B, F = x_cat.shape
    _, V, D = E.shape
    Et = jnp.transpose(E, (0, 2, 1))
    xcatT = x_cat.T
    embT = _sc_pane_gather(xcatT, Et, B, F, D, V)

    nd = x_num.shape[1]
    w1n, w1e = W1[:nd], W1[nd:]
    wab = jnp.concatenate([WA, WB], axis=1)
    bab = jnp.concatenate([bA, bB])[:, None]
    out = _mlp_t(x_num.T, embT, w1n.T, w1e.T, b1[:, None], W2.T, b2[:, None],
                 W3.T, b3[:, None], wab.T, bab)
    return out[0], out[1]


# speedup vs baseline: 1.6179x; 1.0589x over previous
import functools

import jax
import jax.numpy as jnp
from jax import lax
from jax.experimental import pallas as pl
from jax.experimental.pallas import tpu as pltpu
from jax.experimental.pallas import tpu_sc as plsc


def _sc_pane_gather(xcatT, Et, B, F, D, V):
    info = plsc.get_sparse_core_info()
    NC, NS = info.num_cores, info.num_subcores
    NW = NC * NS
    P = F * D
    per_w = P // NW
    CH = 8192
    NCH = B // CH

    mesh = plsc.VectorSubcoreMesh(core_axis_name="c", subcore_axis_name="s")

    @functools.partial(
        pl.kernel,
        mesh=mesh,
        compiler_params=pltpu.CompilerParams(needs_layout_passes=False),
        out_type=jax.ShapeDtypeStruct((P, B), jnp.float32),
        scratch_types=[
            pltpu.VMEM((V,), jnp.float32),
            pltpu.VMEM((CH,), jnp.int32),
            pltpu.VMEM((2, CH), jnp.float32),
            pltpu.SemaphoreType.DMA,
            pltpu.SemaphoreType.DMA,
        ],
    )
    def pane_kernel(xcatT_hbm, et_hbm, out_hbm, row_v, idx_v, out_v, rsem, wsem):
        wid = lax.axis_index("s") * NC + lax.axis_index("c")

        def pair_loop(pi, carry):
            p = wid * per_w + pi
            f = p // D
            d = p % D
            rcp = pltpu.async_copy(et_hbm.at[f, d], row_v, rsem)
            pltpu.sync_copy(xcatT_hbm.at[f, pl.ds(0, CH)], idx_v)
            rcp.wait()

            def chunk_loop(h, carry2):
                b = h % 2

                @pl.when(pi * NCH + h >= 2)
                def _():
                    pltpu.make_async_copy(
                        out_v.at[0], out_hbm.at[0, pl.ds(0, CH)], wsem).wait()

                @plsc.parallel_loop(0, CH, step=16, unroll=8)
                def _(o):
                    idx16 = idx_v[pl.ds(o, 16)]
                    out_v[b, pl.ds(o, 16)] = plsc.load_gather(row_v, [idx16])

                pltpu.async_copy(out_v.at[b], out_hbm.at[p, pl.ds(h * CH, CH)], wsem)

                @pl.when(h + 1 < NCH)
                def _():
                    pltpu.sync_copy(
                        xcatT_hbm.at[f, pl.ds((h + 1) * CH, CH)], idx_v)

                return carry2

            lax.fori_loop(0, NCH, chunk_loop, 0)
            return carry

        lax.fori_loop(0, per_w, pair_loop, 0)
        for _ in range(2):
            pltpu.make_async_copy(
                out_v.at[0], out_hbm.at[0, pl.ds(0, CH)], wsem).wait()

    return pane_kernel(xcatT, Et)


def _mlp_body(xn_ref, emb_ref, w1n_ref, w1e_ref, b1_ref, w2_ref, b2_ref,
              w3_ref, b3_ref, wab_ref, bab_ref, out_ref):
    h = jnp.dot(w1e_ref[...], emb_ref[...], preferred_element_type=jnp.float32)
    h = h + jnp.dot(w1n_ref[...], xn_ref[...], preferred_element_type=jnp.float32)
    h = jnp.maximum(h + b1_ref[...], 0.0)
    h = jnp.maximum(
        jnp.dot(w2_ref[...], h, preferred_element_type=jnp.float32) + b2_ref[...], 0.0)
    h = jnp.maximum(
        jnp.dot(w3_ref[...], h, preferred_element_type=jnp.float32) + b3_ref[...], 0.0)
    out_ref[...] = jnp.dot(wab_ref[...], h, preferred_element_type=jnp.float32) + bab_ref[...]


def _mlp_t(xnT, embT, w1nT, w1eT, b1, w2T, b2, w3T, b3, wabT, bab):
    ED, B = embT.shape
    ND = xnT.shape[0]
    H1, H2, H3 = w2T.shape[1], w3T.shape[1], wabT.shape[1]
    BM = 2048
    grid = (B // BM,)
    full = lambda shape: pl.BlockSpec(shape, lambda i: (0, 0))
    return pl.pallas_call(
        _mlp_body,
        grid=grid,
        in_specs=[
            pl.BlockSpec((ND, BM), lambda i: (0, i)),
            pl.BlockSpec((ED, BM), lambda i: (0, i)),
            full((H1, ND)),
            full((H1, ED)),
            full((H1, 1)),
            full((H2, H1)),
            full((H2, 1)),
            full((H3, H2)),
            full((H3, 1)),
            full((2, H3)),
            full((2, 1)),
        ],
        out_specs=pl.BlockSpec((2, BM), lambda i: (0, i)),
        out_shape=jax.ShapeDtypeStruct((2, B), jnp.float32),
    )(xnT, embT, w1nT, w1eT, b1, w2T, b2, w3T, b3, wabT, bab)


def kernel(x_num, x_cat, E, W1, b1, W2, b2, W3, b3, WA, bA, WB, bB):
    B, F = x_cat.shape
    _, V, D = E.shape
    Et = jnp.transpose(E, (0, 2, 1))
    xcatT = x_cat.T
    embT = _sc_pane_gather(xcatT, Et, B, F, D, V)

    nd = x_num.shape[1]
    w1n, w1e = W1[:nd], W1[nd:]
    wab = jnp.concatenate([WA, WB], axis=1)
    bab = jnp.concatenate([bA, bB])[:, None]
    out = _mlp_t(x_num.T, embT, w1n.T, w1e.T, b1[:, None], W2.T, b2[:, None],
                 W3.T, b3[:, None], wab.T, bab)
    return out[0], out[1]


# speedup vs baseline: 1.6310x; 1.0081x over previous
import functools

import jax
import jax.numpy as jnp
from jax import lax
from jax.experimental import pallas as pl
from jax.experimental.pallas import tpu as pltpu
from jax.experimental.pallas import tpu_sc as plsc


def _sc_pane_gather(xcatT, Et, B, F, D, V):
    info = plsc.get_sparse_core_info()
    NC, NS = info.num_cores, info.num_subcores
    NW = NC * NS
    P = F * D
    per_w = P // NW
    CH = 8192
    NCH = B // CH

    mesh = plsc.VectorSubcoreMesh(core_axis_name="c", subcore_axis_name="s")

    @functools.partial(
        pl.kernel,
        mesh=mesh,
        compiler_params=pltpu.CompilerParams(needs_layout_passes=False),
        out_type=jax.ShapeDtypeStruct((P, B), jnp.float32),
        scratch_types=[
            pltpu.VMEM((V,), jnp.float32),
            pltpu.VMEM((CH,), jnp.int32),
            pltpu.VMEM((2, CH), jnp.float32),
            pltpu.SemaphoreType.DMA,
            pltpu.SemaphoreType.DMA,
        ],
    )
    def pane_kernel(xcatT_hbm, et_hbm, out_hbm, row_v, idx_v, out_v, rsem, wsem):
        wid = lax.axis_index("s") * NC + lax.axis_index("c")

        def pair_loop(pi, carry):
            p = wid * per_w + pi
            f = p // D
            d = p % D
            rcp = pltpu.async_copy(et_hbm.at[f, d], row_v, rsem)
            pltpu.sync_copy(xcatT_hbm.at[f, pl.ds(0, CH)], idx_v)
            rcp.wait()

            def chunk_loop(h, carry2):
                b = h % 2

                @pl.when(pi * NCH + h >= 2)
                def _():
                    pltpu.make_async_copy(
                        out_v.at[0], out_hbm.at[0, pl.ds(0, CH)], wsem).wait()

                @plsc.parallel_loop(0, CH, step=16, unroll=16)
                def _(o):
                    idx16 = idx_v[pl.ds(o, 16)]
                    out_v[b, pl.ds(o, 16)] = plsc.load_gather(row_v, [idx16])

                pltpu.async_copy(out_v.at[b], out_hbm.at[p, pl.ds(h * CH, CH)], wsem)

                @pl.when(h + 1 < NCH)
                def _():
                    pltpu.sync_copy(
                        xcatT_hbm.at[f, pl.ds((h + 1) * CH, CH)], idx_v)

                return carry2

            lax.fori_loop(0, NCH, chunk_loop, 0)
            return carry

        lax.fori_loop(0, per_w, pair_loop, 0)
        for _ in range(2):
            pltpu.make_async_copy(
                out_v.at[0], out_hbm.at[0, pl.ds(0, CH)], wsem).wait()

    return pane_kernel(xcatT, Et)


def _mlp_body(xn_ref, emb_ref, w1n_ref, w1e_ref, b1_ref, w2_ref, b2_ref,
              w3_ref, b3_ref, wab_ref, bab_ref, out_ref):
    h = jnp.dot(w1e_ref[...], emb_ref[...], preferred_element_type=jnp.float32)
    h = h + jnp.dot(w1n_ref[...], xn_ref[...], preferred_element_type=jnp.float32)
    h = jnp.maximum(h + b1_ref[...], 0.0)
    h = jnp.maximum(
        jnp.dot(w2_ref[...], h, preferred_element_type=jnp.float32) + b2_ref[...], 0.0)
    h = jnp.maximum(
        jnp.dot(w3_ref[...], h, preferred_element_type=jnp.float32) + b3_ref[...], 0.0)
    out_ref[...] = jnp.dot(wab_ref[...], h, preferred_element_type=jnp.float32) + bab_ref[...]


def _mlp_t(xnT, embT, w1nT, w1eT, b1, w2T, b2, w3T, b3, wabT, bab):
    ED, B = embT.shape
    ND = xnT.shape[0]
    H1, H2, H3 = w2T.shape[1], w3T.shape[1], wabT.shape[1]
    BM = 4096
    grid = (B // BM,)
    full = lambda shape: pl.BlockSpec(shape, lambda i: (0, 0))
    return pl.pallas_call(
        _mlp_body,
        grid=grid,
        in_specs=[
            pl.BlockSpec((ND, BM), lambda i: (0, i)),
            pl.BlockSpec((ED, BM), lambda i: (0, i)),
            full((H1, ND)),
            full((H1, ED)),
            full((H1, 1)),
            full((H2, H1)),
            full((H2, 1)),
            full((H3, H2)),
            full((H3, 1)),
            full((2, H3)),
            full((2, 1)),
        ],
        out_specs=pl.BlockSpec((2, BM), lambda i: (0, i)),
        out_shape=jax.ShapeDtypeStruct((2, B), jnp.float32),
    )(xnT, embT, w1nT, w1eT, b1, w2T, b2, w3T, b3, wabT, bab)


def kernel(x_num, x_cat, E, W1, b1, W2, b2, W3, b3, WA, bA, WB, bB):
    B, F = x_cat.shape
    _, V, D = E.shape
    Et = jnp.transpose(E, (0, 2, 1))
    xcatT = x_cat.T
    embT = _sc_pane_gather(xcatT, Et, B, F, D, V)

    nd = x_num.shape[1]
    w1n, w1e = W1[:nd], W1[nd:]
    wab = jnp.concatenate([WA, WB], axis=1)
    bab = jnp.concatenate([bA, bB])[:, None]
    out = _mlp_t(x_num.T, embT, w1n.T, w1e.T, b1[:, None], W2.T, b2[:, None],
                 W3.T, b3[:, None], wab.T, bab)
    return out[0], out[1]
